# baseline (speedup 1.0000x reference)
import functools

import jax
import jax.numpy as jnp
from jax import lax
from jax.experimental import pallas as pl
from jax.experimental.pallas import tpu as pltpu
from jax.experimental.pallas import tpu_sc as plsc

ETA = 10
NC = 2
NS = 16
NW = NC * NS
LANES = 16
CHUNK = 128


def _corruption_indices(triples, ent_size):
    key = jax.random.key(42)
    n = triples.shape[0]
    rep = jnp.tile(triples, (ETA, 1))
    kk1, kk2 = jax.random.split(key)
    keep_subj = jax.random.randint(kk1, (n * ETA,), 0, 2, dtype=jnp.int32)
    keep_obj = 1 - keep_subj
    replacements = jax.random.randint(kk2, (n * ETA,), 0, ent_size, dtype=jnp.int32)
    subjects = keep_subj * rep[:, 0] + keep_obj * replacements
    objects = keep_obj * rep[:, 2] + keep_subj * replacements
    return subjects, rep[:, 1], objects


@functools.partial(jax.jit, static_argnames=("total",))
def _distmult_scores(s_idx, p_idx, o_idx, ent_emb, rel_emb, total):
    k_dim = ent_emb.shape[1]
    per_w = total // NW
    n_chunks = per_w // CHUNK
    mesh = plsc.VectorSubcoreMesh(
        core_axis_name="c", subcore_axis_name="s", num_cores=NC, num_subcores=NS
    )

    @functools.partial(
        pl.kernel,
        out_type=jax.ShapeDtypeStruct((total,), jnp.float32),
        mesh=mesh,
        compiler_params=pltpu.CompilerParams(
            use_tc_tiling_on_sc=False, needs_layout_passes=False
        ),
        scratch_types=[
            pltpu.VMEM((CHUNK,), jnp.int32),
            pltpu.VMEM((CHUNK,), jnp.int32),
            pltpu.VMEM((CHUNK,), jnp.int32),
            pltpu.VMEM((CHUNK, k_dim), jnp.float32),
            pltpu.VMEM((CHUNK, k_dim), jnp.float32),
            pltpu.VMEM((CHUNK, k_dim), jnp.float32),
            pltpu.VMEM((CHUNK,), jnp.float32),
            pltpu.SemaphoreType.DMA,
        ],
    )
    def scorer(s_hbm, p_hbm, o_hbm, ent_hbm, rel_hbm, out_hbm,
               sidx_v, pidx_v, oidx_v, srow_v, prow_v, orow_v, out_v, sem):
        wid = lax.axis_index("s") * NC + lax.axis_index("c")
        base = wid * per_w

        def do_chunk(ci, _):
            off = base + ci * CHUNK
            pltpu.sync_copy(s_hbm.at[pl.ds(off, CHUNK)], sidx_v)
            pltpu.sync_copy(p_hbm.at[pl.ds(off, CHUNK)], pidx_v)
            pltpu.sync_copy(o_hbm.at[pl.ds(off, CHUNK)], oidx_v)
            cp_s = pltpu.async_copy(ent_hbm.at[sidx_v], srow_v, sem)
            cp_p = pltpu.async_copy(rel_hbm.at[pidx_v], prow_v, sem)
            cp_o = pltpu.async_copy(ent_hbm.at[oidx_v], orow_v, sem)
            cp_s.wait()
            cp_p.wait()
            cp_o.wait()

            def do_group(g, _):
                rows = g * LANES + lax.iota(jnp.int32, 16)
                acc = jnp.zeros((16,), jnp.float32)
                for k in range(k_dim):
                    kv = jnp.full((16,), k, jnp.int32)
                    e_s = plsc.load_gather(srow_v, [rows, kv])
                    e_p = plsc.load_gather(prow_v, [rows, kv])
                    e_o = plsc.load_gather(orow_v, [rows, kv])
                    acc = acc + e_s * e_p * e_o
                out_v[pl.ds(g * LANES, 16)] = acc
                return _

            lax.fori_loop(0, CHUNK // LANES, do_group, None)
            pltpu.sync_copy(out_v, out_hbm.at[pl.ds(off, CHUNK)])
            return _

        lax.fori_loop(0, n_chunks, do_chunk, None)

    return scorer(s_idx, p_idx, o_idx, ent_emb, rel_emb)


def kernel(inputs, ent_emb, rel_emb):
    n = inputs.shape[0]
    subj, rel, obj = _corruption_indices(inputs, ent_emb.shape[0])
    s_idx = jnp.concatenate([inputs[:, 0], subj])
    p_idx = jnp.concatenate([inputs[:, 1], rel])
    o_idx = jnp.concatenate([inputs[:, 2], obj])
    total = n * (1 + ETA)
    scores = _distmult_scores(s_idx, p_idx, o_idx, ent_emb, rel_emb, total)
    return scores[:n], scores[n:]


# speedup vs baseline: 1.1620x; 1.1620x over previous
import functools

import jax
import jax.numpy as jnp
from jax import lax
from jax.experimental import pallas as pl
from jax.experimental.pallas import tpu as pltpu
from jax.experimental.pallas import tpu_sc as plsc

ETA = 10
NC = 2
NS = 16
NW = NC * NS
LANES = 16
C = 128


def _corruption_indices(triples, ent_size):
    key = jax.random.key(42)
    n = triples.shape[0]
    rep = jnp.tile(triples, (ETA, 1))
    kk1, kk2 = jax.random.split(key)
    keep_subj = jax.random.randint(kk1, (n * ETA,), 0, 2, dtype=jnp.int32)
    keep_obj = 1 - keep_subj
    replacements = jax.random.randint(kk2, (n * ETA,), 0, ent_size, dtype=jnp.int32)
    subjects = keep_subj * rep[:, 0] + keep_obj * replacements
    objects = keep_obj * rep[:, 2] + keep_subj * replacements
    return subjects, rep[:, 1], objects


@functools.partial(jax.jit, static_argnames=("total",))
def _distmult_scores(s_idx, p_idx, o_idx, ent_pad, rel_pad, total):
    k_dim = 64
    kp = ent_pad.shape[1]
    per_w = total // NW
    n_chunks = per_w // C
    n_pairs = n_chunks // 2
    mesh = plsc.VectorSubcoreMesh(
        core_axis_name="c", subcore_axis_name="s", num_cores=NC, num_subcores=NS
    )

    @functools.partial(
        pl.kernel,
        out_type=jax.ShapeDtypeStruct((total,), jnp.float32),
        mesh=mesh,
        compiler_params=pltpu.CompilerParams(
            use_tc_tiling_on_sc=False, needs_layout_passes=False
        ),
        scratch_types=[
            pltpu.VMEM((per_w,), jnp.int32),
            pltpu.VMEM((per_w,), jnp.int32),
            pltpu.VMEM((per_w,), jnp.int32),
            pltpu.VMEM((C, kp), jnp.float32),
            pltpu.VMEM((C, kp), jnp.float32),
            pltpu.VMEM((C, kp), jnp.float32),
            pltpu.VMEM((C, kp), jnp.float32),
            pltpu.VMEM((C, kp), jnp.float32),
            pltpu.VMEM((C, kp), jnp.float32),
            pltpu.VMEM((C,), jnp.float32),
            pltpu.VMEM((C,), jnp.float32),
            pltpu.SemaphoreType.DMA,
            pltpu.SemaphoreType.DMA,
            pltpu.SemaphoreType.DMA,
            pltpu.SemaphoreType.DMA,
        ],
    )
    def scorer(s_hbm, p_hbm, o_hbm, ent_hbm, rel_hbm, out_hbm,
               s_all, p_all, o_all, sb0, sb1, ob0, ob1, pb0, pb1,
               ov0, ov1, gsem0, gsem1, wsem0, wsem1):
        sb = (sb0, sb1)
        ob = (ob0, ob1)
        pb = (pb0, pb1)
        ov = (ov0, ov1)
        gsem = (gsem0, gsem1)
        wsem = (wsem0, wsem1)
        wid = lax.axis_index("s") * NC + lax.axis_index("c")
        base = wid * per_w

        pltpu.sync_copy(s_hbm.at[pl.ds(base, per_w)], s_all)
        pltpu.sync_copy(p_hbm.at[pl.ds(base, per_w)], p_all)
        pltpu.sync_copy(o_hbm.at[pl.ds(base, per_w)], o_all)

        def fire(c, b):
            off = c * C
            pltpu.async_copy(ent_hbm.at[s_all.at[pl.ds(off, C)]], sb[b], gsem[b])
            pltpu.async_copy(ent_hbm.at[o_all.at[pl.ds(off, C)]], ob[b], gsem[b])
            pltpu.async_copy(rel_hbm.at[p_all.at[pl.ds(off, C)]], pb[b], gsem[b])

        def wait_gathers(c, b):
            off = c * C
            pltpu.make_async_copy(
                ent_hbm.at[s_all.at[pl.ds(off, C)]], sb[b], gsem[b]).wait()
            pltpu.make_async_copy(
                ent_hbm.at[o_all.at[pl.ds(off, C)]], ob[b], gsem[b]).wait()
            pltpu.make_async_copy(
                rel_hbm.at[p_all.at[pl.ds(off, C)]], pb[b], gsem[b]).wait()

        fire(0, 0)
        fire(1, 1)

        def pair(i, _):
            for b in range(2):
                c = 2 * i + b
                wait_gathers(c, b)

                @pl.when(c >= 2)
                def _():
                    pltpu.make_async_copy(
                        ov[b], out_hbm.at[pl.ds(base + (c - 2) * C, C)],
                        wsem[b]).wait()

                def grp(g, _):
                    rows = g * LANES + lax.iota(jnp.int32, 16)
                    acc = jnp.zeros((16,), jnp.float32)
                    for k in range(k_dim):
                        kv = jnp.full((16,), k, jnp.int32)
                        e_s = plsc.load_gather(sb[b], [rows, kv])
                        e_p = plsc.load_gather(pb[b], [rows, kv])
                        e_o = plsc.load_gather(ob[b], [rows, kv])
                        acc = acc + e_s * e_p * e_o
                    ov[b][pl.ds(g * LANES, 16)] = acc
                    return _

                lax.fori_loop(0, C // LANES, grp, None)
                pltpu.async_copy(
                    ov[b], out_hbm.at[pl.ds(base + c * C, C)], wsem[b])

                @pl.when(c + 2 < n_chunks)
                def _():
                    fire(c + 2, b)

            return _

        lax.fori_loop(0, n_pairs, pair, None)
        for b in range(2):
            c_last = n_chunks - 2 + b
            pltpu.make_async_copy(
                ov[b], out_hbm.at[pl.ds(base + c_last * C, C)], wsem[b]).wait()

    return scorer(s_idx, p_idx, o_idx, ent_pad, rel_pad)


def kernel(inputs, ent_emb, rel_emb):
    n = inputs.shape[0]
    subj, rel, obj = _corruption_indices(inputs, ent_emb.shape[0])
    s_idx = jnp.concatenate([inputs[:, 0], subj])
    p_idx = jnp.concatenate([inputs[:, 1], rel])
    o_idx = jnp.concatenate([inputs[:, 2], obj])
    ent_pad = jnp.pad(ent_emb, ((0, 0), (0, 128 - ent_emb.shape[1])))
    rel_pad = jnp.pad(rel_emb, ((0, 0), (0, 128 - rel_emb.shape[1])))
    total = n * (1 + ETA)
    scores = _distmult_scores(s_idx, p_idx, o_idx, ent_pad, rel_pad, total)
    return scores[:n], scores[n:]


# speedup vs baseline: 1.2344x; 1.0623x over previous
import functools

import jax
import jax.numpy as jnp
from jax import lax
from jax.experimental import pallas as pl
from jax.experimental.pallas import tpu as pltpu
from jax.experimental.pallas import tpu_sc as plsc

ETA = 10
NC = 2
NS = 16
NW = NC * NS
LANES = 16
C = 64
SPLIT = 32


def _corruption_indices(triples, ent_size):
    key = jax.random.key(42)
    n = triples.shape[0]
    rep = jnp.tile(triples, (ETA, 1))
    kk1, kk2 = jax.random.split(key)
    keep_subj = jax.random.randint(kk1, (n * ETA,), 0, 2, dtype=jnp.int32)
    keep_obj = 1 - keep_subj
    replacements = jax.random.randint(kk2, (n * ETA,), 0, ent_size, dtype=jnp.int32)
    subjects = keep_subj * rep[:, 0] + keep_obj * replacements
    objects = keep_obj * rep[:, 2] + keep_subj * replacements
    return subjects, rep[:, 1], objects


@functools.partial(jax.jit, static_argnames=("total",))
def _distmult_scores(s_idx, p_idx, o_idx, ent_pad, rel_emb, total):
    k_dim = rel_emb.shape[1]
    n_rel = rel_emb.shape[0]
    kp = ent_pad.shape[1]
    per_w = total // NW
    n_chunks = per_w // C
    n_pairs = n_chunks // 2
    mesh = plsc.VectorSubcoreMesh(
        core_axis_name="c", subcore_axis_name="s", num_cores=NC, num_subcores=NS
    )

    @functools.partial(
        pl.kernel,
        out_type=jax.ShapeDtypeStruct((total,), jnp.float32),
        mesh=mesh,
        compiler_params=pltpu.CompilerParams(
            use_tc_tiling_on_sc=False, needs_layout_passes=False
        ),
        scratch_types=[
            pltpu.VMEM((per_w,), jnp.int32),
            pltpu.VMEM((per_w,), jnp.int32),
            pltpu.VMEM((per_w,), jnp.int32),
            pltpu.VMEM((n_rel, k_dim), jnp.float32),
            pltpu.VMEM((C, kp), jnp.float32),
            pltpu.VMEM((C, kp), jnp.float32),
            pltpu.VMEM((C, kp), jnp.float32),
            pltpu.VMEM((C, kp), jnp.float32),
            pltpu.VMEM((C,), jnp.float32),
            pltpu.VMEM((C,), jnp.float32),
            pltpu.SemaphoreType.DMA,
            pltpu.SemaphoreType.DMA,
            pltpu.SemaphoreType.DMA,
            pltpu.SemaphoreType.DMA,
        ],
    )
    def scorer(s_hbm, p_hbm, o_hbm, ent_hbm, rel_hbm, out_hbm,
               s_all, p_all, o_all, rel_v, sb0, sb1, ob0, ob1,
               ov0, ov1, gsem0, gsem1, wsem0, wsem1):
        sb = (sb0, sb1)
        ob = (ob0, ob1)
        ov = (ov0, ov1)
        gsem = (gsem0, gsem1)
        wsem = (wsem0, wsem1)
        wid = lax.axis_index("s") * NC + lax.axis_index("c")
        base = wid * per_w

        pltpu.sync_copy(rel_hbm, rel_v)
        pltpu.sync_copy(s_hbm.at[pl.ds(base, per_w)], s_all)
        pltpu.sync_copy(p_hbm.at[pl.ds(base, per_w)], p_all)
        pltpu.sync_copy(o_hbm.at[pl.ds(base, per_w)], o_all)

        def each_stream(c, b, f):
            off = c * C
            for j in range(C // SPLIT):
                f(ent_hbm.at[s_all.at[pl.ds(off + j * SPLIT, SPLIT)]],
                  sb[b].at[pl.ds(j * SPLIT, SPLIT), :], gsem[b])
                f(ent_hbm.at[o_all.at[pl.ds(off + j * SPLIT, SPLIT)]],
                  ob[b].at[pl.ds(j * SPLIT, SPLIT), :], gsem[b])

        def fire(c, b):
            each_stream(c, b, pltpu.async_copy)

        def wait_gathers(c, b):
            each_stream(c, b,
                        lambda s, d, m: pltpu.make_async_copy(s, d, m).wait())

        fire(0, 0)
        fire(1, 1)

        def pair(i, _):
            for b in range(2):
                c = 2 * i + b
                wait_gathers(c, b)

                @pl.when(c >= 2)
                def _():
                    pltpu.make_async_copy(
                        ov[b], out_hbm.at[pl.ds(base + (c - 2) * C, C)],
                        wsem[b]).wait()

                def grp(g, _):
                    rows = g * LANES + lax.iota(jnp.int32, 16)
                    pv = p_all[pl.ds(c * C + g * LANES, 16)]
                    acc = jnp.zeros((16,), jnp.float32)
                    for k in range(k_dim):
                        kv = jnp.full((16,), k, jnp.int32)
                        e_s = plsc.load_gather(sb[b], [rows, kv])
                        e_p = plsc.load_gather(rel_v, [pv, kv])
                        e_o = plsc.load_gather(ob[b], [rows, kv])
                        acc = acc + e_s * e_p * e_o
                    ov[b][pl.ds(g * LANES, 16)] = acc
                    return _

                lax.fori_loop(0, C // LANES, grp, None)
                pltpu.async_copy(
                    ov[b], out_hbm.at[pl.ds(base + c * C, C)], wsem[b])

                @pl.when(c + 2 < n_chunks)
                def _():
                    fire(c + 2, b)

            return _

        lax.fori_loop(0, n_pairs, pair, None)
        for b in range(2):
            c_last = n_chunks - 2 + b
            pltpu.make_async_copy(
                ov[b], out_hbm.at[pl.ds(base + c_last * C, C)], wsem[b]).wait()

    return scorer(s_idx, p_idx, o_idx, ent_pad, rel_emb)


def kernel(inputs, ent_emb, rel_emb):
    n = inputs.shape[0]
    subj, rel, obj = _corruption_indices(inputs, ent_emb.shape[0])
    s_idx = jnp.concatenate([inputs[:, 0], subj])
    p_idx = jnp.concatenate([inputs[:, 1], rel])
    o_idx = jnp.concatenate([inputs[:, 2], obj])
    ent_pad = jnp.pad(ent_emb, ((0, 0), (0, 128 - ent_emb.shape[1])))
    total = n * (1 + ETA)
    scores = _distmult_scores(s_idx, p_idx, o_idx, ent_pad, rel_emb, total)
    return scores[:n], scores[n:]
